# Initial kernel scaffold; baseline (speedup 1.0000x reference)
import jax
import jax.numpy as jnp
from jax.experimental import pallas as pl


def kernel(x, edge_index, W1, b1, W2, b2):
    raise NotImplementedError("write your pallas kernel here")


# speedup vs baseline: 14.7772x; 14.7772x over previous
import jax
import jax.numpy as jnp
from jax import lax
from jax.experimental import pallas as pl
from jax.experimental.pallas import tpu as pltpu
from jax.experimental.pallas import tpu_sc as plsc

NC = 2
NS = 16
NW = NC * NS
CH = 80
NPAD = 10240


def _sc_mesh():
    return plsc.VectorSubcoreMesh(core_axis_name="c", subcore_axis_name="s")


def _deg_sc(col, zcol):
    (E,) = col.shape
    EW = E // NW
    nch = EW // CH
    rpt = NPAD // NS

    def body(col_hbm, z_hbm, out_hbm, cidx, ones_v, acc):
        c = lax.axis_index("c")
        s = lax.axis_index("s")
        base = (c * NS + s) * EW
        for k in range(CH // 16):
            ones_v[pl.ds(k * 16, 16)] = jnp.ones((16,), jnp.float32)
        pltpu.sync_copy(z_hbm.at[pl.ds(s * rpt, rpt)],
                        acc.at[pl.ds(s * rpt, rpt)])
        plsc.subcore_barrier()

        def step(i, carry):
            pltpu.sync_copy(col_hbm.at[pl.ds(base + i * CH, CH)], cidx)
            pltpu.sync_copy(ones_v, acc.at[cidx], add=True)
            return carry

        lax.fori_loop(0, nch, step, 0)
        plsc.subcore_barrier()
        pltpu.sync_copy(acc.at[pl.ds(s * rpt, rpt)],
                        out_hbm.at[c, pl.ds(s * rpt, rpt)])

    return pl.kernel(
        body,
        out_type=jax.ShapeDtypeStruct((NC, NPAD), jnp.float32),
        mesh=_sc_mesh(),
        scratch_types=[
            pltpu.VMEM((CH,), jnp.int32),
            pltpu.VMEM((CH,), jnp.float32),
            pltpu.VMEM_SHARED((NPAD,), jnp.float32),
        ],
    )(col, zcol)


def _scatter_sc(g, row, col, z2):
    N, D = g.shape
    (E,) = row.shape
    EW = E // NW
    nch = EW // CH
    rpt = NPAD // NS

    def body(g_hbm, row_hbm, col_hbm, z_hbm, out_hbm, ridx, cidx, rows, acc, sem):
        c = lax.axis_index("c")
        s = lax.axis_index("s")
        base = (c * NS + s) * EW
        pltpu.sync_copy(z_hbm.at[pl.ds(s * rpt, rpt)],
                        acc.at[pl.ds(s * rpt, rpt)])
        plsc.subcore_barrier()

        def step(i, carry):
            pltpu.sync_copy(row_hbm.at[pl.ds(base + i * CH, CH)], ridx)
            pltpu.sync_copy(col_hbm.at[pl.ds(base + i * CH, CH)], cidx)
            pltpu.async_copy(g_hbm.at[ridx], rows, sem).wait()
            pltpu.sync_copy(rows, acc.at[cidx], add=True)
            return carry

        lax.fori_loop(0, nch, step, 0)
        plsc.subcore_barrier()
        pltpu.sync_copy(acc.at[pl.ds(s * rpt, rpt)],
                        out_hbm.at[c, pl.ds(s * rpt, rpt)])

    return pl.kernel(
        body,
        out_type=jax.ShapeDtypeStruct((NC, NPAD, D), jnp.float32),
        mesh=_sc_mesh(),
        scratch_types=[
            pltpu.VMEM((CH,), jnp.int32),
            pltpu.VMEM((CH,), jnp.int32),
            pltpu.VMEM((CH, D), jnp.float32),
            pltpu.VMEM_SHARED((NPAD, D), jnp.float32),
            pltpu.SemaphoreType.DMA,
        ],
        compiler_params=pltpu.CompilerParams(use_tc_tiling_on_sc=False),
    )(g, row, col, z2)


def _tc1(x, W1, dega, degb):
    N, IN = x.shape
    H = W1.shape[1]
    R = 2000
    grid = N // R

    def body(x_ref, w_ref, da_ref, db_ref, h_ref, g_ref, dinv_ref):
        deg = da_ref[...] + db_ref[...] + 1.0
        dinv = lax.rsqrt(deg)
        h = jnp.dot(x_ref[...], w_ref[...], preferred_element_type=jnp.float32)
        h_ref[...] = h
        g_ref[...] = h * dinv
        dinv_ref[...] = dinv

    return pl.pallas_call(
        body,
        grid=(grid,),
        in_specs=[
            pl.BlockSpec((R, IN), lambda i: (i, 0)),
            pl.BlockSpec((IN, H), lambda i: (0, 0)),
            pl.BlockSpec((R, 1), lambda i: (i, 0)),
            pl.BlockSpec((R, 1), lambda i: (i, 0)),
        ],
        out_specs=[
            pl.BlockSpec((R, H), lambda i: (i, 0)),
            pl.BlockSpec((R, H), lambda i: (i, 0)),
            pl.BlockSpec((R, 1), lambda i: (i, 0)),
        ],
        out_shape=[
            jax.ShapeDtypeStruct((N, H), jnp.float32),
            jax.ShapeDtypeStruct((N, H), jnp.float32),
            jax.ShapeDtypeStruct((N, 1), jnp.float32),
        ],
    )(x, W1, dega, degb)


def _tc2(s0p, h0, dinv, b1, W2):
    N, H = h0.shape
    O = W2.shape[1]
    R = 2000
    grid = N // R

    def body(sp_ref, h0_ref, dinv_ref, b1_ref, w_ref, y_ref, g_ref):
        dinv = dinv_ref[...]
        t = ((sp_ref[0] + sp_ref[1]) * dinv
             + h0_ref[...] * (dinv * dinv) + b1_ref[...])
        h1 = jnp.maximum(t, 0.0)
        y = jnp.dot(h1, w_ref[...], preferred_element_type=jnp.float32)
        y_ref[...] = y
        g_ref[...] = y * dinv

    return pl.pallas_call(
        body,
        grid=(grid,),
        in_specs=[
            pl.BlockSpec((NC, R, H), lambda i: (0, i, 0)),
            pl.BlockSpec((R, H), lambda i: (i, 0)),
            pl.BlockSpec((R, 1), lambda i: (i, 0)),
            pl.BlockSpec((H,), lambda i: (0,)),
            pl.BlockSpec((H, O), lambda i: (0, 0)),
        ],
        out_specs=[
            pl.BlockSpec((R, O), lambda i: (i, 0)),
            pl.BlockSpec((R, O), lambda i: (i, 0)),
        ],
        out_shape=[
            jax.ShapeDtypeStruct((N, O), jnp.float32),
            jax.ShapeDtypeStruct((N, O), jnp.float32),
        ],
    )(s0p, h0, dinv, b1, W2)


def _tc3(s1p, y1, dinv, b2):
    N, O = y1.shape
    R = 2000
    grid = N // R

    def body(sp_ref, y_ref, dinv_ref, b2_ref, o_ref):
        dinv = dinv_ref[...]
        o_ref[...] = ((sp_ref[0] + sp_ref[1]) * dinv
                      + y_ref[...] * (dinv * dinv) + b2_ref[...])

    return pl.pallas_call(
        body,
        grid=(grid,),
        in_specs=[
            pl.BlockSpec((NC, R, O), lambda i: (0, i, 0)),
            pl.BlockSpec((R, O), lambda i: (i, 0)),
            pl.BlockSpec((R, 1), lambda i: (i, 0)),
            pl.BlockSpec((O,), lambda i: (0,)),
        ],
        out_specs=pl.BlockSpec((R, O), lambda i: (i, 0)),
        out_shape=jax.ShapeDtypeStruct((N, O), jnp.float32),
    )(s1p, y1, dinv, b2)


def kernel(x, edge_index, W1, b1, W2, b2):
    row = edge_index[0].astype(jnp.int32)
    col = edge_index[1].astype(jnp.int32)
    H = W1.shape[1]
    zcol = jnp.zeros((NPAD,), jnp.float32)
    z2 = jnp.zeros((NPAD, H), jnp.float32)

    degp = _deg_sc(col, zcol)
    dega = degp[0].reshape(NPAD, 1)
    degb = degp[1].reshape(NPAD, 1)
    h0, g0, dinv = _tc1(x, W1, dega, degb)
    s0p = _scatter_sc(g0, row, col, z2)
    y1, g1 = _tc2(s0p, h0, dinv, b1, W2)
    s1p = _scatter_sc(g1, row, col, z2)
    out = _tc3(s1p, y1, dinv, b2)
    return out
